# baseline (speedup 1.0000x reference)
import functools

import jax
import jax.numpy as jnp
from jax import lax
from jax.experimental import pallas as pl
from jax.experimental.pallas import tpu as pltpu
from jax.experimental.pallas import tpu_sc as plsc

N = 10000
E = 320000
D = 128
L = 3

NC = 2
NS = 16
NW = NC * NS
EW = E // NW
C = 40
STEPS = EW // C
CD = 80
SD = EW // CD
R8 = (N // NS) // 8 * 8
TAIL = N - NS * R8
DEGW = 128

_mesh = plsc.VectorSubcoreMesh(core_axis_name="c", subcore_axis_name="s")


def _zero_rows(ref, nrows, width):
    def body(i, _):
        for j in range(width // 16):
            ref[i, pl.ds(j * 16, 16)] = jnp.zeros((16,), jnp.float32)
        return 0
    lax.fori_loop(0, nrows, body, 0)


def _zero_shared(zbuf, zrows, acc, sid, sem):
    row0 = pl.multiple_of(sid * R8, 8)
    nfull = R8 // zrows
    for k in range(nfull):
        pltpu.async_copy(zbuf.at[pl.ds(0, zrows)],
                         acc.at[pl.ds(row0 + k * zrows, zrows)], sem)
    rem = R8 - nfull * zrows
    if rem:
        pltpu.async_copy(zbuf.at[pl.ds(0, rem)],
                         acc.at[pl.ds(row0 + nfull * zrows, rem)], sem)

    @pl.when(sid == NS - 1)
    def _():
        pltpu.async_copy(zbuf.at[pl.ds(0, TAIL)],
                         acc.at[pl.ds(NS * R8, TAIL)], sem)

    for k in range(nfull):
        pltpu.make_async_copy(zbuf.at[pl.ds(0, zrows)],
                              acc.at[pl.ds(row0, zrows)], sem).wait()
    if rem:
        pltpu.make_async_copy(zbuf.at[pl.ds(0, rem)],
                              acc.at[pl.ds(row0, rem)], sem).wait()

    @pl.when(sid == NS - 1)
    def _():
        pltpu.make_async_copy(zbuf.at[pl.ds(0, TAIL)],
                              acc.at[pl.ds(row0, TAIL)], sem).wait()


def _write_shared(acc, out_h, cid, sid):
    row0 = pl.multiple_of(sid * R8, 8)
    out0 = pl.multiple_of(cid * N + row0, 8)
    pltpu.sync_copy(acc.at[pl.ds(row0, R8)], out_h.at[pl.ds(out0, R8)])

    @pl.when(sid == NS - 1)
    def _():
        pltpu.sync_copy(acc.at[pl.ds(NS * R8, TAIL)],
                        out_h.at[pl.ds(pl.multiple_of(cid * N + NS * R8, 8), TAIL)])


@functools.partial(
    pl.kernel,
    out_type=jax.ShapeDtypeStruct((NC * N, D), jnp.float32),
    mesh=_mesh,
    scratch_types=[
        pltpu.VMEM((2, 2 * C), jnp.int32),
        pltpu.VMEM((2, 2 * C, D), jnp.float32),
        pltpu.VMEM((2, C, D), jnp.float32),
        pltpu.VMEM_SHARED((N, D), jnp.float32),
        pltpu.SemaphoreType.DMA,
        pltpu.SemaphoreType.DMA,
        pltpu.SemaphoreType.DMA,
        pltpu.SemaphoreType.DMA,
        pltpu.SemaphoreType.DMA,
        pltpu.SemaphoreType.DMA,
    ],
)
def _edge_pass(t_h, pe_h, idx_h, out_h,
               ibuf, rg, re_, acc, si0, si1, sg0, sg1, se0, se1):
    cid = lax.axis_index("c")
    sid = lax.axis_index("s")
    wid = sid * NC + cid
    isems = (si0, si1)
    gsems = (sg0, sg1)
    esems = (se0, se1)

    _zero_rows(rg.at[0], C, D)
    _zero_shared(rg.at[0], C, acc, sid, sg0)
    plsc.subcore_barrier()

    row0 = wid * STEPS
    base0 = wid * EW

    def fire_idx(g, b):
        pltpu.async_copy(idx_h.at[row0 + g], ibuf.at[b], isems[b])

    def wait_idx(b):
        pltpu.make_async_copy(idx_h.at[0], ibuf.at[b], isems[b]).wait()

    def fire_data(g, b):
        pltpu.async_copy(t_h.at[ibuf.at[b]], rg.at[b], gsems[b])
        base = pl.multiple_of(base0 + g * C, 8)
        pltpu.async_copy(pe_h.at[pl.ds(base, C)], re_.at[b], esems[b])

    def consume(b):
        pltpu.make_async_copy(t_h.at[ibuf.at[b]], rg.at[b], gsems[b]).wait()
        pltpu.make_async_copy(pe_h.at[pl.ds(0, C)], re_.at[b], esems[b]).wait()
        rgb, reb = rg.at[b], re_.at[b]

        def crow(i, _):
            for j in range(D // 16):
                sl = pl.ds(j * 16, 16)
                rgb[i, sl] = jnp.maximum(rgb[i, sl] + rgb[C + i, sl] + reb[i, sl], 0.0)
            return 0
        lax.fori_loop(0, C, crow, 0)
        pltpu.sync_copy(rg.at[b, pl.ds(0, C)],
                        acc.at[ibuf.at[b, pl.ds(0, C)]], add=True)

    fire_idx(0, 0)
    fire_idx(1, 1)
    wait_idx(0)
    fire_data(0, 0)

    def pair(k, _):
        g = 2 * k
        wait_idx(1)
        fire_data(g + 1, 1)
        consume(0)
        fire_idx(g + 2, 0)
        wait_idx(0)
        fire_data(g + 2, 0)
        consume(1)
        fire_idx(g + 3, 1)
        return 0

    lax.fori_loop(0, STEPS // 2 - 1, pair, 0)
    wait_idx(1)
    fire_data(STEPS - 1, 1)
    consume(0)
    consume(1)
    plsc.subcore_barrier()
    _write_shared(acc, out_h, cid, sid)


@functools.partial(
    pl.kernel,
    out_type=jax.ShapeDtypeStruct((NC * N, DEGW), jnp.float32),
    mesh=_mesh,
    scratch_types=[
        pltpu.VMEM((SD, CD), jnp.int32),
        pltpu.VMEM((CD, DEGW), jnp.float32),
        pltpu.VMEM_SHARED((N, DEGW), jnp.float32),
        pltpu.SemaphoreType.DMA,
    ],
)
def _deg_pass(dst_h, out_h, dst2, ones_v, acc, sem):
    cid = lax.axis_index("c")
    sid = lax.axis_index("s")
    wid = sid * NC + cid

    _zero_rows(ones_v, CD, DEGW)
    _zero_shared(ones_v, CD, acc, sid, sem)
    pltpu.sync_copy(dst_h.at[wid], dst2)
    plsc.subcore_barrier()

    def fill(i, _):
        for j in range(DEGW // 16):
            ones_v[i, pl.ds(j * 16, 16)] = jnp.ones((16,), jnp.float32)
        return 0
    lax.fori_loop(0, CD, fill, 0)

    def step(g, _):
        pltpu.sync_copy(ones_v, acc.at[dst2.at[g]], add=True)
        return 0

    lax.fori_loop(0, SD, step, 0)
    plsc.subcore_barrier()
    _write_shared(acc, out_h, cid, sid)


BN = 2000
BE = 4000


def _input_body(x_ref, w_ref, b_ref, h_ref):
    h_ref[...] = jnp.dot(x_ref[...], w_ref[...],
                         preferred_element_type=jnp.float32) + b_ref[...]


def _input_proj(x, in_W, in_b):
    return pl.pallas_call(
        _input_body,
        grid=(N // BN,),
        in_specs=[
            pl.BlockSpec((BN, D), lambda i: (i, 0)),
            pl.BlockSpec((D, D), lambda i: (0, 0)),
            pl.BlockSpec((1, D), lambda i: (0, 0)),
        ],
        out_specs=pl.BlockSpec((BN, D), lambda i: (i, 0)),
        out_shape=jax.ShapeDtypeStruct((N, D), jnp.float32),
    )(x, in_W, in_b)


def _project_body(h_ref, w_ref, t_ref):
    t_ref[...] = jnp.dot(h_ref[...], w_ref[0],
                         preferred_element_type=jnp.float32)[None]


def _project(h, wab):
    return pl.pallas_call(
        _project_body,
        grid=(2, N // BN),
        in_specs=[
            pl.BlockSpec((BN, D), lambda j, i: (i, 0)),
            pl.BlockSpec((1, D, D), lambda j, i: (j, 0, 0)),
        ],
        out_specs=pl.BlockSpec((1, BN, D), lambda j, i: (j, i, 0)),
        out_shape=jax.ShapeDtypeStruct((2, N, D), jnp.float32),
    )(h, wab)


def _edgepre_body(ea_ref, wc_ref, b_ref, o_ref):
    o_ref[...] = jnp.dot(ea_ref[...], wc_ref[...],
                         preferred_element_type=jnp.float32) + b_ref[...]


def _edgepre(ea, wc, b1):
    return pl.pallas_call(
        _edgepre_body,
        grid=(E // BE,),
        in_specs=[
            pl.BlockSpec((BE, 4), lambda i: (i, 0)),
            pl.BlockSpec((4, D), lambda i: (0, 0)),
            pl.BlockSpec((1, D), lambda i: (0, 0)),
        ],
        out_specs=pl.BlockSpec((BE, D), lambda i: (i, 0)),
        out_shape=jax.ShapeDtypeStruct((E, D), jnp.float32),
    )(ea, wc, b1)


def _update_body(h_ref, a2_ref, d2_ref, w2_ref, b2_ref, u1_ref, ub1_ref,
                 u2_ref, ub2_ref, g_ref, bb_ref, o_ref):
    h = h_ref[...]
    deg = d2_ref[0, :, 0:1] + d2_ref[1, :, 0:1]
    m = jnp.dot(a2_ref[0] + a2_ref[1], w2_ref[...],
                preferred_element_type=jnp.float32) + deg * b2_ref[...]
    t = jnp.dot(h, u1_ref[0:D], preferred_element_type=jnp.float32)
    t = t + jnp.dot(m, u1_ref[D:2 * D], preferred_element_type=jnp.float32)
    t = jnp.maximum(t + ub1_ref[...], 0.0)
    hn = jnp.dot(t, u2_ref[...], preferred_element_type=jnp.float32) + ub2_ref[...]
    z = hn + h
    mu = jnp.mean(z, axis=-1, keepdims=True)
    zc = z - mu
    var = jnp.mean(zc * zc, axis=-1, keepdims=True)
    o_ref[...] = zc * lax.rsqrt(var + 1e-5) * g_ref[...] + bb_ref[...]


def _update(h, a2, d2, w2, b2, u1, ub1, u2, ub2, g, b):
    return pl.pallas_call(
        _update_body,
        grid=(N // BN,),
        in_specs=[
            pl.BlockSpec((BN, D), lambda i: (i, 0)),
            pl.BlockSpec((2, BN, D), lambda i: (0, i, 0)),
            pl.BlockSpec((2, BN, DEGW), lambda i: (0, i, 0)),
            pl.BlockSpec((D, D), lambda i: (0, 0)),
            pl.BlockSpec((1, D), lambda i: (0, 0)),
            pl.BlockSpec((2 * D, D), lambda i: (0, 0)),
            pl.BlockSpec((1, D), lambda i: (0, 0)),
            pl.BlockSpec((D, D), lambda i: (0, 0)),
            pl.BlockSpec((1, D), lambda i: (0, 0)),
            pl.BlockSpec((1, D), lambda i: (0, 0)),
            pl.BlockSpec((1, D), lambda i: (0, 0)),
        ],
        out_specs=pl.BlockSpec((BN, D), lambda i: (i, 0)),
        out_shape=jax.ShapeDtypeStruct((N, D), jnp.float32),
    )(h, a2, d2, w2, b2, u1, ub1, u2, ub2, g, b)


def kernel(x, edge_index, edge_attr, in_W, in_b, msg_W1, msg_b1, msg_W2, msg_b2,
           upd_W1, upd_b1, upd_W2, upd_b2, ln_g, ln_b):
    src = edge_index[0]
    dst = edge_index[1]
    idx2 = jnp.concatenate([dst.reshape(NW * STEPS, C),
                            src.reshape(NW * STEPS, C) + N], axis=1)
    dst3 = dst.reshape(NW, SD, CD)

    d2 = _deg_pass(dst3).reshape(2, N, DEGW)
    h = _input_proj(x, in_W, in_b.reshape(1, D))

    for l in range(L):
        t2 = _project(h, msg_W1[l, :2 * D].reshape(2, D, D)).reshape(2 * N, D)
        pe = _edgepre(edge_attr, msg_W1[l, 2 * D:], msg_b1[l].reshape(1, D))
        a2 = _edge_pass(t2, pe, idx2).reshape(2, N, D)
        h = _update(h, a2, d2, msg_W2[l], msg_b2[l].reshape(1, D),
                    upd_W1[l], upd_b1[l].reshape(1, D),
                    upd_W2[l], upd_b2[l].reshape(1, D),
                    ln_g[l].reshape(1, D), ln_b[l].reshape(1, D))
    return h


# speedup vs baseline: 1.0538x; 1.0538x over previous
import functools

import jax
import jax.numpy as jnp
from jax import lax
from jax.experimental import pallas as pl
from jax.experimental.pallas import tpu as pltpu
from jax.experimental.pallas import tpu_sc as plsc

N = 10000
E = 320000
D = 128
L = 3

NC = 2
NS = 16
NW = NC * NS
EW = E // NW
C = 40
STEPS = EW // C
CD = 80
SD = EW // CD
R8 = (N // NS) // 8 * 8
TAIL = N - NS * R8
DEGW = 128

_mesh = plsc.VectorSubcoreMesh(core_axis_name="c", subcore_axis_name="s")


def _zero_rows(ref, nrows, width):
    def body(i, _):
        for j in range(width // 16):
            ref[i, pl.ds(j * 16, 16)] = jnp.zeros((16,), jnp.float32)
        return 0
    lax.fori_loop(0, nrows, body, 0)


def _zero_shared(zbuf, zrows, acc, sid, sem):
    row0 = pl.multiple_of(sid * R8, 8)
    nfull = R8 // zrows
    for k in range(nfull):
        pltpu.async_copy(zbuf.at[pl.ds(0, zrows)],
                         acc.at[pl.ds(row0 + k * zrows, zrows)], sem)
    rem = R8 - nfull * zrows
    if rem:
        pltpu.async_copy(zbuf.at[pl.ds(0, rem)],
                         acc.at[pl.ds(row0 + nfull * zrows, rem)], sem)

    @pl.when(sid == NS - 1)
    def _():
        pltpu.async_copy(zbuf.at[pl.ds(0, TAIL)],
                         acc.at[pl.ds(NS * R8, TAIL)], sem)

    for k in range(nfull):
        pltpu.make_async_copy(zbuf.at[pl.ds(0, zrows)],
                              acc.at[pl.ds(row0, zrows)], sem).wait()
    if rem:
        pltpu.make_async_copy(zbuf.at[pl.ds(0, rem)],
                              acc.at[pl.ds(row0, rem)], sem).wait()

    @pl.when(sid == NS - 1)
    def _():
        pltpu.make_async_copy(zbuf.at[pl.ds(0, TAIL)],
                              acc.at[pl.ds(row0, TAIL)], sem).wait()


def _write_shared(acc, out_h, cid, sid):
    row0 = pl.multiple_of(sid * R8, 8)
    out0 = pl.multiple_of(cid * N + row0, 8)
    pltpu.sync_copy(acc.at[pl.ds(row0, R8)], out_h.at[pl.ds(out0, R8)])

    @pl.when(sid == NS - 1)
    def _():
        pltpu.sync_copy(acc.at[pl.ds(NS * R8, TAIL)],
                        out_h.at[pl.ds(pl.multiple_of(cid * N + NS * R8, 8), TAIL)])


@functools.partial(
    pl.kernel,
    out_type=jax.ShapeDtypeStruct((NC * N, D), jnp.float32),
    mesh=_mesh,
    scratch_types=[
        pltpu.VMEM((2, 2 * C), jnp.int32),
        pltpu.VMEM((2, 2 * C, D), jnp.float32),
        pltpu.VMEM((2, C, D), jnp.float32),
        pltpu.VMEM_SHARED((N, D), jnp.float32),
        pltpu.SemaphoreType.DMA,
        pltpu.SemaphoreType.DMA,
        pltpu.SemaphoreType.DMA,
        pltpu.SemaphoreType.DMA,
        pltpu.SemaphoreType.DMA,
        pltpu.SemaphoreType.DMA,
    ],
)
def _edge_pass(t_h, pe_h, idx_h, out_h,
               ibuf, rg, re_, acc, si0, si1, sg0, sg1, se0, se1):
    cid = lax.axis_index("c")
    sid = lax.axis_index("s")
    wid = sid * NC + cid
    isems = (si0, si1)
    gsems = (sg0, sg1)
    esems = (se0, se1)

    _zero_rows(rg.at[0], C, D)
    _zero_shared(rg.at[0], C, acc, sid, sg0)
    plsc.subcore_barrier()

    row0 = wid * STEPS
    base0 = wid * EW

    def fire_idx(g, b):
        pltpu.async_copy(idx_h.at[row0 + g], ibuf.at[b], isems[b])

    def wait_idx(b):
        pltpu.make_async_copy(idx_h.at[0], ibuf.at[b], isems[b]).wait()

    def fire_data(g, b):
        pltpu.async_copy(t_h.at[ibuf.at[b]], rg.at[b], gsems[b])
        base = pl.multiple_of(base0 + g * C, 8)
        pltpu.async_copy(pe_h.at[pl.ds(base, C)], re_.at[b], esems[b])

    def consume(b):
        pltpu.make_async_copy(t_h.at[ibuf.at[b]], rg.at[b], gsems[b]).wait()
        pltpu.make_async_copy(pe_h.at[pl.ds(0, C)], re_.at[b], esems[b]).wait()
        rgb, reb = rg.at[b], re_.at[b]

        @plsc.parallel_loop(0, C, 1, unroll=2)
        def crow(i):
            for j in range(D // 16):
                sl = pl.ds(j * 16, 16)
                rgb[i, sl] = jnp.maximum(rgb[i, sl] + rgb[C + i, sl] + reb[i, sl], 0.0)
        pltpu.sync_copy(rg.at[b, pl.ds(0, C)],
                        acc.at[ibuf.at[b, pl.ds(0, C)]], add=True)

    fire_idx(0, 0)
    fire_idx(1, 1)
    wait_idx(0)
    fire_data(0, 0)

    def pair(k, _):
        g = 2 * k
        wait_idx(1)
        fire_data(g + 1, 1)
        consume(0)
        fire_idx(g + 2, 0)
        wait_idx(0)
        fire_data(g + 2, 0)
        consume(1)
        fire_idx(g + 3, 1)
        return 0

    lax.fori_loop(0, STEPS // 2 - 1, pair, 0)
    wait_idx(1)
    fire_data(STEPS - 1, 1)
    consume(0)
    consume(1)
    plsc.subcore_barrier()
    _write_shared(acc, out_h, cid, sid)


@functools.partial(
    pl.kernel,
    out_type=jax.ShapeDtypeStruct((NC * N, DEGW), jnp.float32),
    mesh=_mesh,
    scratch_types=[
        pltpu.VMEM((SD, CD), jnp.int32),
        pltpu.VMEM((CD, DEGW), jnp.float32),
        pltpu.VMEM_SHARED((N, DEGW), jnp.float32),
        pltpu.SemaphoreType.DMA,
    ],
)
def _deg_pass(dst_h, out_h, dst2, ones_v, acc, sem):
    cid = lax.axis_index("c")
    sid = lax.axis_index("s")
    wid = sid * NC + cid

    _zero_rows(ones_v, CD, DEGW)
    _zero_shared(ones_v, CD, acc, sid, sem)
    pltpu.sync_copy(dst_h.at[wid], dst2)
    plsc.subcore_barrier()

    def fill(i, _):
        for j in range(DEGW // 16):
            ones_v[i, pl.ds(j * 16, 16)] = jnp.ones((16,), jnp.float32)
        return 0
    lax.fori_loop(0, CD, fill, 0)

    def step(g, _):
        pltpu.sync_copy(ones_v, acc.at[dst2.at[g]], add=True)
        return 0

    lax.fori_loop(0, SD, step, 0)
    plsc.subcore_barrier()
    _write_shared(acc, out_h, cid, sid)


BN = 2000
BE = 4000


def _input_body(x_ref, w_ref, b_ref, h_ref):
    h_ref[...] = jnp.dot(x_ref[...], w_ref[...],
                         preferred_element_type=jnp.float32) + b_ref[...]


def _input_proj(x, in_W, in_b):
    return pl.pallas_call(
        _input_body,
        grid=(N // BN,),
        in_specs=[
            pl.BlockSpec((BN, D), lambda i: (i, 0)),
            pl.BlockSpec((D, D), lambda i: (0, 0)),
            pl.BlockSpec((1, D), lambda i: (0, 0)),
        ],
        out_specs=pl.BlockSpec((BN, D), lambda i: (i, 0)),
        out_shape=jax.ShapeDtypeStruct((N, D), jnp.float32),
    )(x, in_W, in_b)


def _project_body(h_ref, w_ref, t_ref):
    t_ref[...] = jnp.dot(h_ref[...], w_ref[0],
                         preferred_element_type=jnp.float32)[None]


def _project(h, wab):
    return pl.pallas_call(
        _project_body,
        grid=(2, N // BN),
        in_specs=[
            pl.BlockSpec((BN, D), lambda j, i: (i, 0)),
            pl.BlockSpec((1, D, D), lambda j, i: (j, 0, 0)),
        ],
        out_specs=pl.BlockSpec((1, BN, D), lambda j, i: (j, i, 0)),
        out_shape=jax.ShapeDtypeStruct((2, N, D), jnp.float32),
    )(h, wab)


def _edgepre_body(ea_ref, wc_ref, b_ref, o_ref):
    o_ref[...] = jnp.dot(ea_ref[...], wc_ref[...],
                         preferred_element_type=jnp.float32) + b_ref[...]


def _edgepre(ea, wc, b1):
    return pl.pallas_call(
        _edgepre_body,
        grid=(E // BE,),
        in_specs=[
            pl.BlockSpec((BE, 4), lambda i: (i, 0)),
            pl.BlockSpec((4, D), lambda i: (0, 0)),
            pl.BlockSpec((1, D), lambda i: (0, 0)),
        ],
        out_specs=pl.BlockSpec((BE, D), lambda i: (i, 0)),
        out_shape=jax.ShapeDtypeStruct((E, D), jnp.float32),
    )(ea, wc, b1)


def _update_body(h_ref, a2_ref, d2_ref, w2_ref, b2_ref, u1_ref, ub1_ref,
                 u2_ref, ub2_ref, g_ref, bb_ref, o_ref):
    h = h_ref[...]
    deg = d2_ref[...]
    m = jnp.dot(a2_ref[0] + a2_ref[1], w2_ref[...],
                preferred_element_type=jnp.float32) + deg * b2_ref[...]
    t = jnp.dot(h, u1_ref[0:D], preferred_element_type=jnp.float32)
    t = t + jnp.dot(m, u1_ref[D:2 * D], preferred_element_type=jnp.float32)
    t = jnp.maximum(t + ub1_ref[...], 0.0)
    hn = jnp.dot(t, u2_ref[...], preferred_element_type=jnp.float32) + ub2_ref[...]
    z = hn + h
    mu = jnp.mean(z, axis=-1, keepdims=True)
    zc = z - mu
    var = jnp.mean(zc * zc, axis=-1, keepdims=True)
    o_ref[...] = zc * lax.rsqrt(var + 1e-5) * g_ref[...] + bb_ref[...]


def _update(h, a2, d2, w2, b2, u1, ub1, u2, ub2, g, b):
    return pl.pallas_call(
        _update_body,
        grid=(N // BN,),
        in_specs=[
            pl.BlockSpec((BN, D), lambda i: (i, 0)),
            pl.BlockSpec((2, BN, D), lambda i: (0, i, 0)),
            pl.BlockSpec((BN, 1), lambda i: (i, 0)),
            pl.BlockSpec((D, D), lambda i: (0, 0)),
            pl.BlockSpec((1, D), lambda i: (0, 0)),
            pl.BlockSpec((2 * D, D), lambda i: (0, 0)),
            pl.BlockSpec((1, D), lambda i: (0, 0)),
            pl.BlockSpec((D, D), lambda i: (0, 0)),
            pl.BlockSpec((1, D), lambda i: (0, 0)),
            pl.BlockSpec((1, D), lambda i: (0, 0)),
            pl.BlockSpec((1, D), lambda i: (0, 0)),
        ],
        out_specs=pl.BlockSpec((BN, D), lambda i: (i, 0)),
        out_shape=jax.ShapeDtypeStruct((N, D), jnp.float32),
    )(h, a2, d2, w2, b2, u1, ub1, u2, ub2, g, b)


def _updproj_body(h_ref, a2_ref, d2_ref, w2_ref, b2_ref, u1_ref, ub1_ref,
                  u2_ref, ub2_ref, g_ref, bb_ref, wab_ref, o_ref, t_ref):
    h = h_ref[...]
    deg = d2_ref[...]
    m = jnp.dot(a2_ref[0] + a2_ref[1], w2_ref[...],
                preferred_element_type=jnp.float32) + deg * b2_ref[...]
    t = jnp.dot(h, u1_ref[0:D], preferred_element_type=jnp.float32)
    t = t + jnp.dot(m, u1_ref[D:2 * D], preferred_element_type=jnp.float32)
    t = jnp.maximum(t + ub1_ref[...], 0.0)
    hn = jnp.dot(t, u2_ref[...], preferred_element_type=jnp.float32) + ub2_ref[...]
    z = hn + h
    mu = jnp.mean(z, axis=-1, keepdims=True)
    zc = z - mu
    var = jnp.mean(zc * zc, axis=-1, keepdims=True)
    hnew = zc * lax.rsqrt(var + 1e-5) * g_ref[...] + bb_ref[...]
    o_ref[...] = hnew
    t_ref[0] = jnp.dot(hnew, wab_ref[0], preferred_element_type=jnp.float32)
    t_ref[1] = jnp.dot(hnew, wab_ref[1], preferred_element_type=jnp.float32)


def _update_proj(h, a2, d2, w2, b2, u1, ub1, u2, ub2, g, b, wab):
    full = lambda s: pl.BlockSpec(s, lambda i: tuple(0 for _ in s))
    return pl.pallas_call(
        _updproj_body,
        grid=(N // BN,),
        in_specs=[
            pl.BlockSpec((BN, D), lambda i: (i, 0)),
            pl.BlockSpec((2, BN, D), lambda i: (0, i, 0)),
            pl.BlockSpec((BN, 1), lambda i: (i, 0)),
            full((D, D)),
            full((1, D)),
            full((2 * D, D)),
            full((1, D)),
            full((D, D)),
            full((1, D)),
            full((1, D)),
            full((1, D)),
            full((2, D, D)),
        ],
        out_specs=[
            pl.BlockSpec((BN, D), lambda i: (i, 0)),
            pl.BlockSpec((2, BN, D), lambda i: (0, i, 0)),
        ],
        out_shape=[
            jax.ShapeDtypeStruct((N, D), jnp.float32),
            jax.ShapeDtypeStruct((2, N, D), jnp.float32),
        ],
    )(h, a2, d2, w2, b2, u1, ub1, u2, ub2, g, b, wab)


def kernel(x, edge_index, edge_attr, in_W, in_b, msg_W1, msg_b1, msg_W2, msg_b2,
           upd_W1, upd_b1, upd_W2, upd_b2, ln_g, ln_b):
    src = edge_index[0]
    dst = edge_index[1]
    idx2 = jnp.concatenate([dst.reshape(NW * STEPS, C),
                            src.reshape(NW * STEPS, C) + N], axis=1)
    dst3 = dst.reshape(NW, SD, CD)

    d2 = _deg_pass(dst3)
    deg = d2[0:N, 0:1] + d2[N:2 * N, 0:1]
    h = _input_proj(x, in_W, in_b.reshape(1, D))
    t2 = _project(h, msg_W1[0, :2 * D].reshape(2, D, D)).reshape(2 * N, D)

    for l in range(L):
        pe = _edgepre(edge_attr, msg_W1[l, 2 * D:], msg_b1[l].reshape(1, D))
        a2 = _edge_pass(t2, pe, idx2).reshape(2, N, D)
        args = (h, a2, deg, msg_W2[l], msg_b2[l].reshape(1, D),
                upd_W1[l], upd_b1[l].reshape(1, D),
                upd_W2[l], upd_b2[l].reshape(1, D),
                ln_g[l].reshape(1, D), ln_b[l].reshape(1, D))
        if l < L - 1:
            h, t2n = _update_proj(*args, msg_W1[l + 1, :2 * D].reshape(2, D, D))
            t2 = t2n.reshape(2 * N, D)
        else:
            h = _update(*args)
    return h
